# baseline (speedup 1.0000x reference)
import functools
from math import sqrt

import jax
import jax.numpy as jnp
import numpy as np
from jax.experimental import pallas as pl
from jax.experimental.pallas import tpu as pltpu

_RADIUS = 9.0
_LEAK = 0.2
_EPS = 1e-5
_BI = 128
_BJ = 128
_PADJ = 1024
_CUTS = 8
_HCH = 16


def _leaky(x):
    return jnp.where(x >= 0, x, _LEAK * x)


def _pre_body(featT, nuv9, P, M, Sp, W1, b1, W2, b2, g, be, Cb1,
              FT_out, G_out, C_out, *, n_real):
    np_ = featT.shape[1]
    t1 = _leaky(jnp.dot(W1[...], featT[...],
                        preferred_element_type=jnp.float32) + b1[...])
    t2 = _leaky(jnp.dot(W2[...], t1,
                        preferred_element_type=jnp.float32) + b2[...])
    mask = (jax.lax.broadcasted_iota(jnp.int32, (1, np_), 1)
            < n_real).astype(jnp.float32)
    t2 = t2 * mask
    den = 4.0 * n_real
    groups = []
    for gi in range(4):
        sub = t2[4 * gi:4 * gi + 4, :]
        s1 = jnp.sum(sub, keepdims=True)
        s2 = jnp.sum(sub * sub, keepdims=True)
        mean = s1 / den
        var = s2 / den - mean * mean
        groups.append((sub - mean) * jax.lax.rsqrt(var + _EPS))
    norm = jnp.concatenate(groups, axis=0)
    FT_out[...] = (norm * g[...] + be[...]) * mask
    Gc = jnp.dot(nuv9[...], M[...], preferred_element_type=jnp.float32)
    G_out[...] = Gc
    Pt = jnp.concatenate([P[...]] * _CUTS, axis=1)
    C_out[...] = Cb1[...] - jnp.dot(Gc * Pt, Sp[...],
                                    preferred_element_type=jnp.float32)


def _conv_body(P, PT, NI, NT, SQ, SQT, G, C, FT, Cw2s, Cb2s, out, acc):
    j = pl.program_id(1)
    nj = pl.num_programs(1)
    bi = P.shape[0]
    bj = PT.shape[1]
    lhs = jnp.concatenate(
        [P[...]] + [G[:, 3 * k:3 * k + 3] for k in range(_CUTS)], axis=0)
    R = jnp.dot(lhs, PT[...], preferred_element_type=jnp.float32)
    ndot = jnp.dot(NI[...], NT[...], preferred_element_type=jnp.float32)
    pij = R[0:bi]
    sumsq = SQ[...] + SQT[...] - 2.0 * pij
    t = 2.0 - ndot
    w = jnp.exp(-(sumsq * t * t))
    Ys = [jnp.maximum(R[(k + 1) * bi:(k + 2) * bi] + C[:, k:k + 1], 0.0)
          for k in range(_CUTS)]

    @pl.when(j == 0)
    def _():
        acc[...] = jnp.zeros_like(acc)

    for d in range(_HCH):
        s = Ys[0] * Cw2s[d, 0]
        for k in range(1, _CUTS):
            s = s + Ys[k] * Cw2s[d, k]
        z = jnp.maximum(s + Cb2s[d, 0], 0.0)
        acc[:, d * bj:(d + 1) * bj] += (w * z) * FT[d:d + 1, :]

    @pl.when(j == nj - 1)
    def _():
        a = acc[...]
        cols = [jnp.sum(a[:, d * bj:(d + 1) * bj], axis=1, keepdims=True)
                for d in range(_HCH)]
        out[...] = jnp.concatenate(cols, axis=1)


def _post_body(X, W1T, b1, W2T, b2, g, be, out, *, n_real):
    np_ = X.shape[0]
    h = _leaky(jnp.dot(X[...], W1T[...],
                       preferred_element_type=jnp.float32) + b1[...])
    h = _leaky(jnp.dot(h, W2T[...],
                       preferred_element_type=jnp.float32) + b2[...])
    rmask = (jax.lax.broadcasted_iota(jnp.int32, (np_, 1), 0)
             < n_real).astype(jnp.float32)
    hm = h * rmask
    och = h.shape[1]
    gch = och // 4
    den = float(gch) * n_real
    groups = []
    for gi in range(4):
        sub = hm[:, gch * gi:gch * gi + gch]
        s1 = jnp.sum(sub, keepdims=True)
        s2 = jnp.sum(sub * sub, keepdims=True)
        mean = s1 / den
        var = s2 / den - mean * mean
        groups.append((sub - mean) * jax.lax.rsqrt(var + _EPS))
    norm = jnp.concatenate(groups, axis=1)
    out[...] = norm * g[...] + be[...]


def kernel(points, nuv, features, W_in1, b_in1, W_in2, b_in2, g_in, be_in,
           Cw1, Cb1, Cw2, Cb2, W_out1, b_out1, W_out2, b_out2, g_out, be_out):
    n = points.shape[0]
    och = W_out1.shape[0]
    np_ = ((n + _PADJ - 1) // _PADJ) * _PADJ
    pad = np_ - n
    f32 = jnp.float32

    p = (points * (1.0 / (sqrt(2.0) * _RADIUS))).astype(f32)
    P = jnp.pad(p, ((0, pad), (0, 0)))
    PT = P.T
    NI = jnp.pad(nuv[:, 0, :], ((0, pad), (0, 0)))
    NT = NI.T
    SQ = jnp.sum(P * P, axis=1, keepdims=True)
    SQT = SQ.T
    nuv9 = jnp.pad(nuv.reshape(n, 9), ((0, pad), (0, 0)))
    featT = jnp.pad(features.T, ((0, 0), (0, pad)))

    rows, colsx, kk, aa = [], [], [], []
    for k in range(_CUTS):
        for a in range(3):
            for b in range(3):
                rows.append(3 * a + b)
                colsx.append(3 * k + b)
                kk.append(k)
                aa.append(a)
    M = jnp.zeros((9, 3 * _CUTS), f32).at[
        jnp.array(rows), jnp.array(colsx)].set(Cw1[jnp.array(kk), jnp.array(aa)])
    Sp_np = np.zeros((3 * _CUTS, _CUTS), np.float32)
    for k in range(_CUTS):
        for b in range(3):
            Sp_np[3 * k + b, k] = 1.0
    Sp = jnp.asarray(Sp_np)

    FT, G, C = pl.pallas_call(
        functools.partial(_pre_body, n_real=n),
        out_shape=[
            jax.ShapeDtypeStruct((_HCH, np_), f32),
            jax.ShapeDtypeStruct((np_, 3 * _CUTS), f32),
            jax.ShapeDtypeStruct((np_, _CUTS), f32),
        ],
    )(featT, nuv9, P, M, Sp, W_in1, b_in1.reshape(-1, 1), W_in2,
      b_in2.reshape(-1, 1), g_in.reshape(-1, 1), be_in.reshape(-1, 1),
      Cb1.reshape(1, -1))

    grid = (np_ // _BI, np_ // _BJ)
    conv = pl.pallas_call(
        _conv_body,
        grid=grid,
        in_specs=[
            pl.BlockSpec((_BI, 3), lambda i, j: (i, 0)),
            pl.BlockSpec((3, _BJ), lambda i, j: (0, j)),
            pl.BlockSpec((_BI, 3), lambda i, j: (i, 0)),
            pl.BlockSpec((3, _BJ), lambda i, j: (0, j)),
            pl.BlockSpec((_BI, 1), lambda i, j: (i, 0)),
            pl.BlockSpec((1, _BJ), lambda i, j: (0, j)),
            pl.BlockSpec((_BI, 3 * _CUTS), lambda i, j: (i, 0)),
            pl.BlockSpec((_BI, _CUTS), lambda i, j: (i, 0)),
            pl.BlockSpec((_HCH, _BJ), lambda i, j: (0, j)),
            pl.BlockSpec(memory_space=pltpu.SMEM),
            pl.BlockSpec(memory_space=pltpu.SMEM),
        ],
        out_specs=pl.BlockSpec((_BI, _HCH), lambda i, j: (i, 0)),
        out_shape=jax.ShapeDtypeStruct((np_, _HCH), f32),
        scratch_shapes=[pltpu.VMEM((_BI, _HCH * _BJ), f32)],
    )(P, PT, NI, NT, SQ, SQT, G, C, FT, Cw2, Cb2.reshape(-1, 1))

    outp = pl.pallas_call(
        functools.partial(_post_body, n_real=n),
        out_shape=jax.ShapeDtypeStruct((np_, och), f32),
    )(conv, W_out1.T, b_out1.reshape(1, -1), W_out2.T,
      b_out2.reshape(1, -1), g_out.reshape(1, -1), be_out.reshape(1, -1))
    return outp[:n]


# speedup vs baseline: 1.4136x; 1.4136x over previous
import functools
from math import sqrt

import jax
import jax.numpy as jnp
import numpy as np
from jax.experimental import pallas as pl
from jax.experimental.pallas import tpu as pltpu

_RADIUS = 9.0
_LEAK = 0.2
_EPS = 1e-5
_BI = 128
_BJ = 1024
_SUB = 128
_PADJ = 1024
_CUTS = 8
_HCH = 16


def _leaky(x):
    return jnp.where(x >= 0, x, _LEAK * x)


def _pre_body(featT, nuv9, P, M, Sp, W1, b1, W2, b2, g, be, Cb1,
              FT_out, G_out, C_out, *, n_real):
    np_ = featT.shape[1]
    t1 = _leaky(jnp.dot(W1[...], featT[...],
                        preferred_element_type=jnp.float32) + b1[...])
    t2 = _leaky(jnp.dot(W2[...], t1,
                        preferred_element_type=jnp.float32) + b2[...])
    mask = (jax.lax.broadcasted_iota(jnp.int32, (1, np_), 1)
            < n_real).astype(jnp.float32)
    t2 = t2 * mask
    den = 4.0 * n_real
    groups = []
    for gi in range(4):
        sub = t2[4 * gi:4 * gi + 4, :]
        s1 = jnp.sum(sub, keepdims=True)
        s2 = jnp.sum(sub * sub, keepdims=True)
        mean = s1 / den
        var = s2 / den - mean * mean
        groups.append((sub - mean) * jax.lax.rsqrt(var + _EPS))
    norm = jnp.concatenate(groups, axis=0)
    FT_out[...] = (norm * g[...] + be[...]) * mask
    Gc = jnp.dot(nuv9[...], M[...], preferred_element_type=jnp.float32)
    G_out[...] = Gc
    Pt = jnp.concatenate([P[...]] * _CUTS, axis=1)
    C_out[...] = Cb1[...] - jnp.dot(Gc * Pt, Sp[...],
                                    preferred_element_type=jnp.float32)


def _conv_body(P, PT, NI, NT, SQ, SQT, G, C, FT, Cw2s, Cb2s, out):
    j = pl.program_id(1)
    bi = P.shape[0]
    bj = PT.shape[1]
    lhs = jnp.concatenate(
        [P[...]] + [G[:, 3 * k:3 * k + 3] for k in range(_CUTS)], axis=0)
    R = jnp.dot(lhs, PT[...], preferred_element_type=jnp.float32)
    ndot = jnp.dot(NI[...], NT[...], preferred_element_type=jnp.float32)
    Cb = [C[:, k:k + 1] for k in range(_CUTS)]
    acc = None
    for c in range(bj // _SUB):
        sl = slice(c * _SUB, (c + 1) * _SUB)
        pij = R[0:bi, sl]
        sumsq = SQ[...] + SQT[:, sl] - 2.0 * pij
        t = 2.0 - ndot[:, sl]
        w = jnp.exp(-(sumsq * t * t))
        Ys = [jnp.maximum(R[(k + 1) * bi:(k + 2) * bi, sl] + Cb[k], 0.0)
              for k in range(_CUTS)]
        cols = []
        for d in range(_HCH):
            s = Ys[0] * Cw2s[d, 0]
            for k in range(1, _CUTS):
                s = s + Ys[k] * Cw2s[d, k]
            z = jnp.maximum(s + Cb2s[d, 0], 0.0)
            cols.append(jnp.sum(w * z * FT[d:d + 1, sl],
                                axis=1, keepdims=True))
        part = jnp.concatenate(cols, axis=1)
        acc = part if acc is None else acc + part

    @pl.when(j == 0)
    def _():
        out[...] = jnp.zeros_like(out)

    out[...] += acc


def _post_body(X, W1T, b1, W2T, b2, g, be, out, *, n_real):
    np_ = X.shape[0]
    h = _leaky(jnp.dot(X[...], W1T[...],
                       preferred_element_type=jnp.float32) + b1[...])
    h = _leaky(jnp.dot(h, W2T[...],
                       preferred_element_type=jnp.float32) + b2[...])
    rmask = (jax.lax.broadcasted_iota(jnp.int32, (np_, 1), 0)
             < n_real).astype(jnp.float32)
    hm = h * rmask
    och = h.shape[1]
    gch = och // 4
    den = float(gch) * n_real
    groups = []
    for gi in range(4):
        sub = hm[:, gch * gi:gch * gi + gch]
        s1 = jnp.sum(sub, keepdims=True)
        s2 = jnp.sum(sub * sub, keepdims=True)
        mean = s1 / den
        var = s2 / den - mean * mean
        groups.append((sub - mean) * jax.lax.rsqrt(var + _EPS))
    norm = jnp.concatenate(groups, axis=1)
    out[...] = norm * g[...] + be[...]


def kernel(points, nuv, features, W_in1, b_in1, W_in2, b_in2, g_in, be_in,
           Cw1, Cb1, Cw2, Cb2, W_out1, b_out1, W_out2, b_out2, g_out, be_out):
    n = points.shape[0]
    och = W_out1.shape[0]
    np_ = ((n + _PADJ - 1) // _PADJ) * _PADJ
    pad = np_ - n
    f32 = jnp.float32

    p = (points * (1.0 / (sqrt(2.0) * _RADIUS))).astype(f32)
    P = jnp.pad(p, ((0, pad), (0, 0)))
    PT = P.T
    NI = jnp.pad(nuv[:, 0, :], ((0, pad), (0, 0)))
    NT = NI.T
    SQ = jnp.sum(P * P, axis=1, keepdims=True)
    SQT = SQ.T
    nuv9 = jnp.pad(nuv.reshape(n, 9), ((0, pad), (0, 0)))
    featT = jnp.pad(features.T, ((0, 0), (0, pad)))

    rows, colsx, kk, aa = [], [], [], []
    for k in range(_CUTS):
        for a in range(3):
            for b in range(3):
                rows.append(3 * a + b)
                colsx.append(3 * k + b)
                kk.append(k)
                aa.append(a)
    M = jnp.zeros((9, 3 * _CUTS), f32).at[
        jnp.array(rows), jnp.array(colsx)].set(Cw1[jnp.array(kk), jnp.array(aa)])
    Sp_np = np.zeros((3 * _CUTS, _CUTS), np.float32)
    for k in range(_CUTS):
        for b in range(3):
            Sp_np[3 * k + b, k] = 1.0
    Sp = jnp.asarray(Sp_np)

    FT, G, C = pl.pallas_call(
        functools.partial(_pre_body, n_real=n),
        out_shape=[
            jax.ShapeDtypeStruct((_HCH, np_), f32),
            jax.ShapeDtypeStruct((np_, 3 * _CUTS), f32),
            jax.ShapeDtypeStruct((np_, _CUTS), f32),
        ],
    )(featT, nuv9, P, M, Sp, W_in1, b_in1.reshape(-1, 1), W_in2,
      b_in2.reshape(-1, 1), g_in.reshape(-1, 1), be_in.reshape(-1, 1),
      Cb1.reshape(1, -1))

    grid = (np_ // _BI, np_ // _BJ)
    conv = pl.pallas_call(
        _conv_body,
        grid=grid,
        in_specs=[
            pl.BlockSpec((_BI, 3), lambda i, j: (i, 0)),
            pl.BlockSpec((3, _BJ), lambda i, j: (0, j)),
            pl.BlockSpec((_BI, 3), lambda i, j: (i, 0)),
            pl.BlockSpec((3, _BJ), lambda i, j: (0, j)),
            pl.BlockSpec((_BI, 1), lambda i, j: (i, 0)),
            pl.BlockSpec((1, _BJ), lambda i, j: (0, j)),
            pl.BlockSpec((_BI, 3 * _CUTS), lambda i, j: (i, 0)),
            pl.BlockSpec((_BI, _CUTS), lambda i, j: (i, 0)),
            pl.BlockSpec((_HCH, _BJ), lambda i, j: (0, j)),
            pl.BlockSpec(memory_space=pltpu.SMEM),
            pl.BlockSpec(memory_space=pltpu.SMEM),
        ],
        out_specs=pl.BlockSpec((_BI, _HCH), lambda i, j: (i, 0)),
        out_shape=jax.ShapeDtypeStruct((np_, _HCH), f32),
    )(P, PT, NI, NT, SQ, SQT, G, C, FT, Cw2, Cb2.reshape(-1, 1))

    outp = pl.pallas_call(
        functools.partial(_post_body, n_real=n),
        out_shape=jax.ShapeDtypeStruct((np_, och), f32),
    )(conv, W_out1.T, b_out1.reshape(1, -1), W_out2.T,
      b_out2.reshape(1, -1), g_out.reshape(1, -1), be_out.reshape(1, -1))
    return outp[:n]


# speedup vs baseline: 1.9303x; 1.3656x over previous
import functools
from math import sqrt

import jax
import jax.numpy as jnp
import numpy as np
from jax.experimental import pallas as pl
from jax.experimental.pallas import tpu as pltpu

_RADIUS = 9.0
_LEAK = 0.2
_EPS = 1e-5
_BI = 128
_BJ = 1024
_SUB = 128
_PADJ = 1024
_CUTS = 8
_HCH = 16


def _leaky(x):
    return jnp.where(x >= 0, x, _LEAK * x)


def _pre_body(featT, nuv9, P, M, Sp, W1, b1, W2, b2, g, be, Cb1,
              FT_out, G_out, C_out, *, n_real):
    np_ = featT.shape[1]
    t1 = _leaky(jnp.dot(W1[...], featT[...],
                        preferred_element_type=jnp.float32) + b1[...])
    t2 = _leaky(jnp.dot(W2[...], t1,
                        preferred_element_type=jnp.float32) + b2[...])
    mask = (jax.lax.broadcasted_iota(jnp.int32, (1, np_), 1)
            < n_real).astype(jnp.float32)
    t2 = t2 * mask
    den = 4.0 * n_real
    groups = []
    for gi in range(4):
        sub = t2[4 * gi:4 * gi + 4, :]
        s1 = jnp.sum(sub, keepdims=True)
        s2 = jnp.sum(sub * sub, keepdims=True)
        mean = s1 / den
        var = s2 / den - mean * mean
        groups.append((sub - mean) * jax.lax.rsqrt(var + _EPS))
    norm = jnp.concatenate(groups, axis=0)
    FT_out[...] = (norm * g[...] + be[...]) * mask
    Gc = jnp.dot(nuv9[...], M[...], preferred_element_type=jnp.float32)
    G_out[...] = Gc
    Pt = jnp.concatenate([P[...]] * _CUTS, axis=1)
    C_out[...] = Cb1[...] - jnp.dot(Gc * Pt, Sp[...],
                                    preferred_element_type=jnp.float32)


def _conv_body(P, PT, NI, NT, SQ, SQT, G, C, FT, Cw2s, Cb2s, out):
    j = pl.program_id(1)
    bi = P.shape[0]
    bj = PT.shape[1]
    lhs = jnp.concatenate(
        [P[...]] + [G[:, 3 * k:3 * k + 3] for k in range(_CUTS)], axis=0)
    R = jnp.dot(lhs, PT[...], preferred_element_type=jnp.float32)
    ndot = jnp.dot(NI[...], NT[...], preferred_element_type=jnp.float32)
    Cb = [C[:, k:k + 1] for k in range(_CUTS)]
    acc = None
    for c in range(bj // _SUB):
        sl = slice(c * _SUB, (c + 1) * _SUB)
        pij = R[0:bi, sl]
        sumsq = SQ[...] + SQT[:, sl] - 2.0 * pij
        t = 2.0 - ndot[:, sl]
        w = jnp.exp(-(sumsq * t * t))
        Ys = [jnp.maximum(R[(k + 1) * bi:(k + 2) * bi, sl] + Cb[k], 0.0)
              .astype(jnp.bfloat16) for k in range(_CUTS)]
        cols = []
        for d in range(_HCH):
            s = Ys[0] * jnp.bfloat16(Cw2s[d, 0])
            for k in range(1, _CUTS):
                s = s + Ys[k] * jnp.bfloat16(Cw2s[d, k])
            z = jnp.maximum(s.astype(jnp.float32) + Cb2s[d, 0], 0.0)
            cols.append(jnp.sum(w * z * FT[d:d + 1, sl],
                                axis=1, keepdims=True))
        part = jnp.concatenate(cols, axis=1)
        acc = part if acc is None else acc + part

    @pl.when(j == 0)
    def _():
        out[...] = jnp.zeros_like(out)

    out[...] += acc


def _post_body(X, W1T, b1, W2T, b2, g, be, out, *, n_real):
    np_ = X.shape[0]
    h = _leaky(jnp.dot(X[...], W1T[...],
                       preferred_element_type=jnp.float32) + b1[...])
    h = _leaky(jnp.dot(h, W2T[...],
                       preferred_element_type=jnp.float32) + b2[...])
    rmask = (jax.lax.broadcasted_iota(jnp.int32, (np_, 1), 0)
             < n_real).astype(jnp.float32)
    hm = h * rmask
    och = h.shape[1]
    gch = och // 4
    den = float(gch) * n_real
    groups = []
    for gi in range(4):
        sub = hm[:, gch * gi:gch * gi + gch]
        s1 = jnp.sum(sub, keepdims=True)
        s2 = jnp.sum(sub * sub, keepdims=True)
        mean = s1 / den
        var = s2 / den - mean * mean
        groups.append((sub - mean) * jax.lax.rsqrt(var + _EPS))
    norm = jnp.concatenate(groups, axis=1)
    out[...] = norm * g[...] + be[...]


def kernel(points, nuv, features, W_in1, b_in1, W_in2, b_in2, g_in, be_in,
           Cw1, Cb1, Cw2, Cb2, W_out1, b_out1, W_out2, b_out2, g_out, be_out):
    n = points.shape[0]
    och = W_out1.shape[0]
    np_ = ((n + _PADJ - 1) // _PADJ) * _PADJ
    pad = np_ - n
    f32 = jnp.float32

    p = (points * (1.0 / (sqrt(2.0) * _RADIUS))).astype(f32)
    P = jnp.pad(p, ((0, pad), (0, 0)))
    PT = P.T
    NI = jnp.pad(nuv[:, 0, :], ((0, pad), (0, 0)))
    NT = NI.T
    SQ = jnp.sum(P * P, axis=1, keepdims=True)
    SQT = SQ.T
    nuv9 = jnp.pad(nuv.reshape(n, 9), ((0, pad), (0, 0)))
    featT = jnp.pad(features.T, ((0, 0), (0, pad)))

    rows, colsx, kk, aa = [], [], [], []
    for k in range(_CUTS):
        for a in range(3):
            for b in range(3):
                rows.append(3 * a + b)
                colsx.append(3 * k + b)
                kk.append(k)
                aa.append(a)
    M = jnp.zeros((9, 3 * _CUTS), f32).at[
        jnp.array(rows), jnp.array(colsx)].set(Cw1[jnp.array(kk), jnp.array(aa)])
    Sp_np = np.zeros((3 * _CUTS, _CUTS), np.float32)
    for k in range(_CUTS):
        for b in range(3):
            Sp_np[3 * k + b, k] = 1.0
    Sp = jnp.asarray(Sp_np)

    FT, G, C = pl.pallas_call(
        functools.partial(_pre_body, n_real=n),
        out_shape=[
            jax.ShapeDtypeStruct((_HCH, np_), f32),
            jax.ShapeDtypeStruct((np_, 3 * _CUTS), f32),
            jax.ShapeDtypeStruct((np_, _CUTS), f32),
        ],
    )(featT, nuv9, P, M, Sp, W_in1, b_in1.reshape(-1, 1), W_in2,
      b_in2.reshape(-1, 1), g_in.reshape(-1, 1), be_in.reshape(-1, 1),
      Cb1.reshape(1, -1))

    grid = (np_ // _BI, np_ // _BJ)
    conv = pl.pallas_call(
        _conv_body,
        grid=grid,
        in_specs=[
            pl.BlockSpec((_BI, 3), lambda i, j: (i, 0)),
            pl.BlockSpec((3, _BJ), lambda i, j: (0, j)),
            pl.BlockSpec((_BI, 3), lambda i, j: (i, 0)),
            pl.BlockSpec((3, _BJ), lambda i, j: (0, j)),
            pl.BlockSpec((_BI, 1), lambda i, j: (i, 0)),
            pl.BlockSpec((1, _BJ), lambda i, j: (0, j)),
            pl.BlockSpec((_BI, 3 * _CUTS), lambda i, j: (i, 0)),
            pl.BlockSpec((_BI, _CUTS), lambda i, j: (i, 0)),
            pl.BlockSpec((_HCH, _BJ), lambda i, j: (0, j)),
            pl.BlockSpec(memory_space=pltpu.SMEM),
            pl.BlockSpec(memory_space=pltpu.SMEM),
        ],
        out_specs=pl.BlockSpec((_BI, _HCH), lambda i, j: (i, 0)),
        out_shape=jax.ShapeDtypeStruct((np_, _HCH), f32),
    )(P, PT, NI, NT, SQ, SQT, G, C, FT, Cw2, Cb2.reshape(-1, 1))

    outp = pl.pallas_call(
        functools.partial(_post_body, n_real=n),
        out_shape=jax.ShapeDtypeStruct((np_, och), f32),
    )(conv, W_out1.T, b_out1.reshape(1, -1), W_out2.T,
      b_out2.reshape(1, -1), g_out.reshape(1, -1), be_out.reshape(1, -1))
    return outp[:n]


# speedup vs baseline: 1.9472x; 1.0087x over previous
import functools
from math import sqrt

import jax
import jax.numpy as jnp
import numpy as np
from jax.experimental import pallas as pl
from jax.experimental.pallas import tpu as pltpu

_RADIUS = 9.0
_LEAK = 0.2
_EPS = 1e-5
_BI = 128
_BJ = 2048
_SUB = 128
_PADJ = 2048
_CUTS = 8
_HCH = 16


def _leaky(x):
    return jnp.where(x >= 0, x, _LEAK * x)


def _pre_body(featT, nuv9, P, M, Sp, W1, b1, W2, b2, g, be, Cb1,
              FT_out, G_out, C_out, *, n_real):
    np_ = featT.shape[1]
    t1 = _leaky(jnp.dot(W1[...], featT[...],
                        preferred_element_type=jnp.float32) + b1[...])
    t2 = _leaky(jnp.dot(W2[...], t1,
                        preferred_element_type=jnp.float32) + b2[...])
    mask = (jax.lax.broadcasted_iota(jnp.int32, (1, np_), 1)
            < n_real).astype(jnp.float32)
    t2 = t2 * mask
    den = 4.0 * n_real
    groups = []
    for gi in range(4):
        sub = t2[4 * gi:4 * gi + 4, :]
        s1 = jnp.sum(sub, keepdims=True)
        s2 = jnp.sum(sub * sub, keepdims=True)
        mean = s1 / den
        var = s2 / den - mean * mean
        groups.append((sub - mean) * jax.lax.rsqrt(var + _EPS))
    norm = jnp.concatenate(groups, axis=0)
    FT_out[...] = (norm * g[...] + be[...]) * mask
    Gc = jnp.dot(nuv9[...], M[...], preferred_element_type=jnp.float32)
    G_out[...] = Gc
    Pt = jnp.concatenate([P[...]] * _CUTS, axis=1)
    C_out[...] = Cb1[...] - jnp.dot(Gc * Pt, Sp[...],
                                    preferred_element_type=jnp.float32)


def _conv_body(P4, PT4, NI4, NT4, SQTn, G4, FT, Cw2s, Cb2s, out):
    j = pl.program_id(1)
    bi = P4.shape[0]
    bj = PT4.shape[1]
    lhs = jnp.concatenate(
        [P4[...]] + [G4[:, 4 * k:4 * k + 4] for k in range(_CUTS)], axis=0)
    R = jnp.dot(lhs, PT4[...], preferred_element_type=jnp.float32)
    tmat = jnp.dot(NI4[...], NT4[...], preferred_element_type=jnp.float32)
    zero_bf = jnp.bfloat16(0)
    acc = None
    for c in range(bj // _SUB):
        sl = slice(c * _SUB, (c + 1) * _SUB)
        neg_sumsq = R[0:bi, sl] + SQTn[:, sl]
        t = tmat[:, sl]
        w = jnp.exp(neg_sumsq * t * t)
        Ys = [jnp.maximum(R[(k + 1) * bi:(k + 2) * bi, sl]
                          .astype(jnp.bfloat16), zero_bf)
              for k in range(_CUTS)]
        cols = []
        for d in range(_HCH):
            s = Ys[0] * jnp.bfloat16(Cw2s[d, 0])
            for k in range(1, _CUTS):
                s = s + Ys[k] * jnp.bfloat16(Cw2s[d, k])
            z = jnp.maximum(s.astype(jnp.float32) + Cb2s[d, 0], 0.0)
            cols.append(jnp.sum(w * z * FT[d:d + 1, sl],
                                axis=1, keepdims=True))
        part = jnp.concatenate(cols, axis=1)
        acc = part if acc is None else acc + part

    @pl.when(j == 0)
    def _():
        out[...] = jnp.zeros_like(out)

    out[...] += acc


def _post_body(X, W1T, b1, W2T, b2, g, be, out, *, n_real):
    np_ = X.shape[0]
    h = _leaky(jnp.dot(X[...], W1T[...],
                       preferred_element_type=jnp.float32) + b1[...])
    h = _leaky(jnp.dot(h, W2T[...],
                       preferred_element_type=jnp.float32) + b2[...])
    rmask = (jax.lax.broadcasted_iota(jnp.int32, (np_, 1), 0)
             < n_real).astype(jnp.float32)
    hm = h * rmask
    och = h.shape[1]
    gch = och // 4
    den = float(gch) * n_real
    groups = []
    for gi in range(4):
        sub = hm[:, gch * gi:gch * gi + gch]
        s1 = jnp.sum(sub, keepdims=True)
        s2 = jnp.sum(sub * sub, keepdims=True)
        mean = s1 / den
        var = s2 / den - mean * mean
        groups.append((sub - mean) * jax.lax.rsqrt(var + _EPS))
    norm = jnp.concatenate(groups, axis=1)
    out[...] = norm * g[...] + be[...]


def kernel(points, nuv, features, W_in1, b_in1, W_in2, b_in2, g_in, be_in,
           Cw1, Cb1, Cw2, Cb2, W_out1, b_out1, W_out2, b_out2, g_out, be_out):
    n = points.shape[0]
    och = W_out1.shape[0]
    np_ = ((n + _PADJ - 1) // _PADJ) * _PADJ
    pad = np_ - n
    f32 = jnp.float32

    p = (points * (1.0 / (sqrt(2.0) * _RADIUS))).astype(f32)
    P = jnp.pad(p, ((0, pad), (0, 0)))
    PT = P.T
    NI = jnp.pad(nuv[:, 0, :], ((0, pad), (0, 0)))
    NT = NI.T
    SQ = jnp.sum(P * P, axis=1, keepdims=True)
    SQT = SQ.T
    nuv9 = jnp.pad(nuv.reshape(n, 9), ((0, pad), (0, 0)))
    featT = jnp.pad(features.T, ((0, 0), (0, pad)))

    rows, colsx, kk, aa = [], [], [], []
    for k in range(_CUTS):
        for a in range(3):
            for b in range(3):
                rows.append(3 * a + b)
                colsx.append(3 * k + b)
                kk.append(k)
                aa.append(a)
    M = jnp.zeros((9, 3 * _CUTS), f32).at[
        jnp.array(rows), jnp.array(colsx)].set(Cw1[jnp.array(kk), jnp.array(aa)])
    Sp_np = np.zeros((3 * _CUTS, _CUTS), np.float32)
    for k in range(_CUTS):
        for b in range(3):
            Sp_np[3 * k + b, k] = 1.0
    Sp = jnp.asarray(Sp_np)

    FT, G, C = pl.pallas_call(
        functools.partial(_pre_body, n_real=n),
        out_shape=[
            jax.ShapeDtypeStruct((_HCH, np_), f32),
            jax.ShapeDtypeStruct((np_, 3 * _CUTS), f32),
            jax.ShapeDtypeStruct((np_, _CUTS), f32),
        ],
    )(featT, nuv9, P, M, Sp, W_in1, b_in1.reshape(-1, 1), W_in2,
      b_in2.reshape(-1, 1), g_in.reshape(-1, 1), be_in.reshape(-1, 1),
      Cb1.reshape(1, -1))

    ones_col = jnp.ones((np_, 1), f32)
    P4 = jnp.concatenate([2.0 * P, -SQ], axis=1)
    PT4 = jnp.concatenate([PT, jnp.ones((1, np_), f32)], axis=0)
    NI4 = jnp.concatenate([-NI, ones_col], axis=1)
    NT4 = jnp.concatenate([NT, 2.0 * jnp.ones((1, np_), f32)], axis=0)
    SQTn = -SQT
    G4 = jnp.concatenate(
        sum([[G[:, 3 * k:3 * k + 3], C[:, k:k + 1]] for k in range(_CUTS)],
            []), axis=1)

    grid = (np_ // _BI, np_ // _BJ)
    conv = pl.pallas_call(
        _conv_body,
        grid=grid,
        in_specs=[
            pl.BlockSpec((_BI, 4), lambda i, j: (i, 0)),
            pl.BlockSpec((4, _BJ), lambda i, j: (0, j)),
            pl.BlockSpec((_BI, 4), lambda i, j: (i, 0)),
            pl.BlockSpec((4, _BJ), lambda i, j: (0, j)),
            pl.BlockSpec((1, _BJ), lambda i, j: (0, j)),
            pl.BlockSpec((_BI, 4 * _CUTS), lambda i, j: (i, 0)),
            pl.BlockSpec((_HCH, _BJ), lambda i, j: (0, j)),
            pl.BlockSpec(memory_space=pltpu.SMEM),
            pl.BlockSpec(memory_space=pltpu.SMEM),
        ],
        out_specs=pl.BlockSpec((_BI, _HCH), lambda i, j: (i, 0)),
        out_shape=jax.ShapeDtypeStruct((np_, _HCH), f32),
    )(P4, PT4, NI4, NT4, SQTn, G4, FT, Cw2, Cb2.reshape(-1, 1))

    outp = pl.pallas_call(
        functools.partial(_post_body, n_real=n),
        out_shape=jax.ShapeDtypeStruct((np_, och), f32),
    )(conv, W_out1.T, b_out1.reshape(1, -1), W_out2.T,
      b_out2.reshape(1, -1), g_out.reshape(1, -1), be_out.reshape(1, -1))
    return outp[:n]


# speedup vs baseline: 2.1941x; 1.1268x over previous
import functools
from math import sqrt

import jax
import jax.numpy as jnp
import numpy as np
from jax.experimental import pallas as pl
from jax.experimental.pallas import tpu as pltpu

_RADIUS = 9.0
_LEAK = 0.2
_EPS = 1e-5
_BI = 128
_BJ = 2048
_SUB = 128
_SUBI = 32
_PADJ = 2048
_CUTS = 8
_HCH = 16


def _leaky(x):
    return jnp.where(x >= 0, x, _LEAK * x)


def _pre_body(featT, nuv9, P, M, Sp, W1, b1, W2, b2, g, be, Cb1,
              FT_out, G_out, C_out, *, n_real):
    np_ = featT.shape[1]
    t1 = _leaky(jnp.dot(W1[...], featT[...],
                        preferred_element_type=jnp.float32) + b1[...])
    t2 = _leaky(jnp.dot(W2[...], t1,
                        preferred_element_type=jnp.float32) + b2[...])
    mask = (jax.lax.broadcasted_iota(jnp.int32, (1, np_), 1)
            < n_real).astype(jnp.float32)
    t2 = t2 * mask
    den = 4.0 * n_real
    groups = []
    for gi in range(4):
        sub = t2[4 * gi:4 * gi + 4, :]
        s1 = jnp.sum(sub, keepdims=True)
        s2 = jnp.sum(sub * sub, keepdims=True)
        mean = s1 / den
        var = s2 / den - mean * mean
        groups.append((sub - mean) * jax.lax.rsqrt(var + _EPS))
    norm = jnp.concatenate(groups, axis=0)
    FT_out[...] = (norm * g[...] + be[...]) * mask
    Gc = jnp.dot(nuv9[...], M[...], preferred_element_type=jnp.float32)
    G_out[...] = Gc
    Pt = jnp.concatenate([P[...]] * _CUTS, axis=1)
    C_out[...] = Cb1[...] - jnp.dot(Gc * Pt, Sp[...],
                                    preferred_element_type=jnp.float32)


def _conv_body(P4, PT4, NI4, NT4, SQTn, G4, FTb, Cw2s, Cb2s, out):
    j = pl.program_id(1)
    bi = P4.shape[0]
    bj = PT4.shape[1]
    lhs = jnp.concatenate(
        [P4[...]] + [G4[:, 4 * k:4 * k + 4] for k in range(_CUTS)], axis=0)
    R = jnp.dot(lhs, PT4[...], preferred_element_type=jnp.float32)
    tmat = jnp.dot(NI4[...], NT4[...], preferred_element_type=jnp.float32)
    zero_bf = jnp.bfloat16(0)
    cw_bf = [[jnp.bfloat16(Cw2s[d, k]) for k in range(_CUTS)]
             for d in range(_HCH)]
    cb_bf = [jnp.bfloat16(Cb2s[d, 0]) for d in range(_HCH)]
    accs = [None] * (bi // _SUBI)
    for c in range(bj // _SUB):
        sl = slice(c * _SUB, (c + 1) * _SUB)
        for ii in range(bi // _SUBI):
            rsl = slice(ii * _SUBI, (ii + 1) * _SUBI)
            neg_sumsq = R[ii * _SUBI:(ii + 1) * _SUBI, sl] + SQTn[:, sl]
            t = tmat[rsl, sl]
            w_bf = jnp.exp(neg_sumsq * t * t).astype(jnp.bfloat16)
            Ys = [jnp.maximum(
                R[(k + 1) * bi + ii * _SUBI:(k + 1) * bi + (ii + 1) * _SUBI,
                  sl].astype(jnp.bfloat16), zero_bf)
                for k in range(_CUTS)]
            cols = []
            for d in range(_HCH):
                s = Ys[0] * cw_bf[d][0]
                for k in range(1, _CUTS):
                    s = s + Ys[k] * cw_bf[d][k]
                z = jnp.maximum(s + cb_bf[d], zero_bf)
                prod = (z * (w_bf * FTb[d:d + 1, sl])).astype(jnp.float32)
                cols.append(jnp.sum(prod, axis=1, keepdims=True))
            part = jnp.concatenate(cols, axis=1)
            accs[ii] = part if accs[ii] is None else accs[ii] + part
    acc = jnp.concatenate(accs, axis=0)

    @pl.when(j == 0)
    def _():
        out[...] = jnp.zeros_like(out)

    out[...] += acc


def _post_body(X, W1T, b1, W2T, b2, g, be, out, *, n_real):
    np_ = X.shape[0]
    h = _leaky(jnp.dot(X[...], W1T[...],
                       preferred_element_type=jnp.float32) + b1[...])
    h = _leaky(jnp.dot(h, W2T[...],
                       preferred_element_type=jnp.float32) + b2[...])
    rmask = (jax.lax.broadcasted_iota(jnp.int32, (np_, 1), 0)
             < n_real).astype(jnp.float32)
    hm = h * rmask
    och = h.shape[1]
    gch = och // 4
    den = float(gch) * n_real
    groups = []
    for gi in range(4):
        sub = hm[:, gch * gi:gch * gi + gch]
        s1 = jnp.sum(sub, keepdims=True)
        s2 = jnp.sum(sub * sub, keepdims=True)
        mean = s1 / den
        var = s2 / den - mean * mean
        groups.append((sub - mean) * jax.lax.rsqrt(var + _EPS))
    norm = jnp.concatenate(groups, axis=1)
    out[...] = norm * g[...] + be[...]


def kernel(points, nuv, features, W_in1, b_in1, W_in2, b_in2, g_in, be_in,
           Cw1, Cb1, Cw2, Cb2, W_out1, b_out1, W_out2, b_out2, g_out, be_out):
    n = points.shape[0]
    och = W_out1.shape[0]
    np_ = ((n + _PADJ - 1) // _PADJ) * _PADJ
    pad = np_ - n
    f32 = jnp.float32

    p = (points * (1.0 / (sqrt(2.0) * _RADIUS))).astype(f32)
    P = jnp.pad(p, ((0, pad), (0, 0)))
    PT = P.T
    NI = jnp.pad(nuv[:, 0, :], ((0, pad), (0, 0)))
    NT = NI.T
    SQ = jnp.sum(P * P, axis=1, keepdims=True)
    SQT = SQ.T
    nuv9 = jnp.pad(nuv.reshape(n, 9), ((0, pad), (0, 0)))
    featT = jnp.pad(features.T, ((0, 0), (0, pad)))

    rows, colsx, kk, aa = [], [], [], []
    for k in range(_CUTS):
        for a in range(3):
            for b in range(3):
                rows.append(3 * a + b)
                colsx.append(3 * k + b)
                kk.append(k)
                aa.append(a)
    M = jnp.zeros((9, 3 * _CUTS), f32).at[
        jnp.array(rows), jnp.array(colsx)].set(Cw1[jnp.array(kk), jnp.array(aa)])
    Sp_np = np.zeros((3 * _CUTS, _CUTS), np.float32)
    for k in range(_CUTS):
        for b in range(3):
            Sp_np[3 * k + b, k] = 1.0
    Sp = jnp.asarray(Sp_np)

    FT, G, C = pl.pallas_call(
        functools.partial(_pre_body, n_real=n),
        out_shape=[
            jax.ShapeDtypeStruct((_HCH, np_), f32),
            jax.ShapeDtypeStruct((np_, 3 * _CUTS), f32),
            jax.ShapeDtypeStruct((np_, _CUTS), f32),
        ],
    )(featT, nuv9, P, M, Sp, W_in1, b_in1.reshape(-1, 1), W_in2,
      b_in2.reshape(-1, 1), g_in.reshape(-1, 1), be_in.reshape(-1, 1),
      Cb1.reshape(1, -1))

    ones_col = jnp.ones((np_, 1), f32)
    P4 = jnp.concatenate([2.0 * P, -SQ], axis=1)
    PT4 = jnp.concatenate([PT, jnp.ones((1, np_), f32)], axis=0)
    NI4 = jnp.concatenate([-NI, ones_col], axis=1)
    NT4 = jnp.concatenate([NT, 2.0 * jnp.ones((1, np_), f32)], axis=0)
    SQTn = -SQT
    G4 = jnp.concatenate(
        sum([[G[:, 3 * k:3 * k + 3], C[:, k:k + 1]] for k in range(_CUTS)],
            []), axis=1)

    grid = (np_ // _BI, np_ // _BJ)
    conv = pl.pallas_call(
        _conv_body,
        grid=grid,
        in_specs=[
            pl.BlockSpec((_BI, 4), lambda i, j: (i, 0)),
            pl.BlockSpec((4, _BJ), lambda i, j: (0, j)),
            pl.BlockSpec((_BI, 4), lambda i, j: (i, 0)),
            pl.BlockSpec((4, _BJ), lambda i, j: (0, j)),
            pl.BlockSpec((1, _BJ), lambda i, j: (0, j)),
            pl.BlockSpec((_BI, 4 * _CUTS), lambda i, j: (i, 0)),
            pl.BlockSpec((_HCH, _BJ), lambda i, j: (0, j)),
            pl.BlockSpec(memory_space=pltpu.SMEM),
            pl.BlockSpec(memory_space=pltpu.SMEM),
        ],
        out_specs=pl.BlockSpec((_BI, _HCH), lambda i, j: (i, 0)),
        out_shape=jax.ShapeDtypeStruct((np_, _HCH), f32),
    )(P4, PT4, NI4, NT4, SQTn, G4, FT.astype(jnp.bfloat16), Cw2,
      Cb2.reshape(-1, 1))

    outp = pl.pallas_call(
        functools.partial(_post_body, n_real=n),
        out_shape=jax.ShapeDtypeStruct((np_, och), f32),
    )(conv, W_out1.T, b_out1.reshape(1, -1), W_out2.T,
      b_out2.reshape(1, -1), g_out.reshape(1, -1), be_out.reshape(1, -1))
    return outp[:n]


# speedup vs baseline: 2.1958x; 1.0008x over previous
import functools
from math import sqrt

import jax
import jax.numpy as jnp
import numpy as np
from jax.experimental import pallas as pl
from jax.experimental.pallas import tpu as pltpu

_RADIUS = 9.0
_LEAK = 0.2
_EPS = 1e-5
_BI = 128
_BJ = 2048
_SUB = 128
_SUBI = 32
_PADJ = 2048
_CUTS = 8
_HCH = 16


def _leaky(x):
    return jnp.where(x >= 0, x, _LEAK * x)


def _pre_body(featT, nuv9, P, M, Sp, W1, b1, W2, b2, g, be, Cb1,
              FT_out, G_out, C_out, *, n_real):
    np_ = featT.shape[1]
    t1 = _leaky(jnp.dot(W1[...], featT[...],
                        preferred_element_type=jnp.float32) + b1[...])
    t2 = _leaky(jnp.dot(W2[...], t1,
                        preferred_element_type=jnp.float32) + b2[...])
    mask = (jax.lax.broadcasted_iota(jnp.int32, (1, np_), 1)
            < n_real).astype(jnp.float32)
    t2 = t2 * mask
    den = 4.0 * n_real
    groups = []
    for gi in range(4):
        sub = t2[4 * gi:4 * gi + 4, :]
        s1 = jnp.sum(sub, keepdims=True)
        s2 = jnp.sum(sub * sub, keepdims=True)
        mean = s1 / den
        var = s2 / den - mean * mean
        groups.append((sub - mean) * jax.lax.rsqrt(var + _EPS))
    norm = jnp.concatenate(groups, axis=0)
    FT_out[...] = (norm * g[...] + be[...]) * mask
    Gc = jnp.dot(nuv9[...], M[...], preferred_element_type=jnp.float32)
    G_out[...] = Gc
    Pt = jnp.concatenate([P[...]] * _CUTS, axis=1)
    C_out[...] = Cb1[...] - jnp.dot(Gc * Pt, Sp[...],
                                    preferred_element_type=jnp.float32)


def _conv_body(P4, PT4, NI4, NT4, SQTn, G4, FTb, Cw2s, Cb2s, out):
    j = pl.program_id(1)
    bi = P4.shape[0]
    bj = PT4.shape[1]
    lhs = jnp.concatenate(
        [P4[...]] + [G4[:, 4 * k:4 * k + 4] for k in range(_CUTS)], axis=0)
    R = jnp.dot(lhs, PT4[...], preferred_element_type=jnp.float32)
    tmat = jnp.dot(NI4[...], NT4[...], preferred_element_type=jnp.float32)
    zero_bf = jnp.bfloat16(0)
    cw_bf = [[jnp.bfloat16(Cw2s[d, k]) for k in range(_CUTS)]
             for d in range(_HCH)]
    cb_bf = [jnp.bfloat16(Cb2s[d, 0]) for d in range(_HCH)]
    accs = [None] * (bi // _SUBI)
    for c in range(bj // _SUB):
        sl = slice(c * _SUB, (c + 1) * _SUB)
        for ii in range(bi // _SUBI):
            rsl = slice(ii * _SUBI, (ii + 1) * _SUBI)
            neg_sumsq = R[ii * _SUBI:(ii + 1) * _SUBI, sl] + SQTn[:, sl]
            t = tmat[rsl, sl]
            w_bf = jnp.exp(neg_sumsq * t * t).astype(jnp.bfloat16)
            Ys = [jnp.maximum(
                R[(k + 1) * bi + ii * _SUBI:(k + 1) * bi + (ii + 1) * _SUBI,
                  sl].astype(jnp.bfloat16), zero_bf)
                for k in range(_CUTS)]
            cols = []
            for d in range(_HCH):
                terms = [Ys[k] * cw_bf[d][k] for k in range(_CUTS)]
                while len(terms) > 1:
                    terms = [terms[t] + terms[t + 1]
                             for t in range(0, len(terms) - 1, 2)] + \
                            (terms[-1:] if len(terms) % 2 else [])
                s = terms[0]
                z = jnp.maximum(s + cb_bf[d], zero_bf)
                prod = (z * (w_bf * FTb[d:d + 1, sl])).astype(jnp.float32)
                cols.append(jnp.sum(prod, axis=1, keepdims=True))
            part = jnp.concatenate(cols, axis=1)
            accs[ii] = part if accs[ii] is None else accs[ii] + part
    acc = jnp.concatenate(accs, axis=0)

    @pl.when(j == 0)
    def _():
        out[...] = jnp.zeros_like(out)

    out[...] += acc


def _post_body(X, W1T, b1, W2T, b2, g, be, out, *, n_real):
    np_ = X.shape[0]
    h = _leaky(jnp.dot(X[...], W1T[...],
                       preferred_element_type=jnp.float32) + b1[...])
    h = _leaky(jnp.dot(h, W2T[...],
                       preferred_element_type=jnp.float32) + b2[...])
    rmask = (jax.lax.broadcasted_iota(jnp.int32, (np_, 1), 0)
             < n_real).astype(jnp.float32)
    hm = h * rmask
    och = h.shape[1]
    gch = och // 4
    den = float(gch) * n_real
    groups = []
    for gi in range(4):
        sub = hm[:, gch * gi:gch * gi + gch]
        s1 = jnp.sum(sub, keepdims=True)
        s2 = jnp.sum(sub * sub, keepdims=True)
        mean = s1 / den
        var = s2 / den - mean * mean
        groups.append((sub - mean) * jax.lax.rsqrt(var + _EPS))
    norm = jnp.concatenate(groups, axis=1)
    out[...] = norm * g[...] + be[...]


def kernel(points, nuv, features, W_in1, b_in1, W_in2, b_in2, g_in, be_in,
           Cw1, Cb1, Cw2, Cb2, W_out1, b_out1, W_out2, b_out2, g_out, be_out):
    n = points.shape[0]
    och = W_out1.shape[0]
    np_ = ((n + _PADJ - 1) // _PADJ) * _PADJ
    pad = np_ - n
    f32 = jnp.float32

    p = (points * (1.0 / (sqrt(2.0) * _RADIUS))).astype(f32)
    P = jnp.pad(p, ((0, pad), (0, 0)))
    PT = P.T
    NI = jnp.pad(nuv[:, 0, :], ((0, pad), (0, 0)))
    NT = NI.T
    SQ = jnp.sum(P * P, axis=1, keepdims=True)
    SQT = SQ.T
    nuv9 = jnp.pad(nuv.reshape(n, 9), ((0, pad), (0, 0)))
    featT = jnp.pad(features.T, ((0, 0), (0, pad)))

    rows, colsx, kk, aa = [], [], [], []
    for k in range(_CUTS):
        for a in range(3):
            for b in range(3):
                rows.append(3 * a + b)
                colsx.append(3 * k + b)
                kk.append(k)
                aa.append(a)
    M = jnp.zeros((9, 3 * _CUTS), f32).at[
        jnp.array(rows), jnp.array(colsx)].set(Cw1[jnp.array(kk), jnp.array(aa)])
    Sp_np = np.zeros((3 * _CUTS, _CUTS), np.float32)
    for k in range(_CUTS):
        for b in range(3):
            Sp_np[3 * k + b, k] = 1.0
    Sp = jnp.asarray(Sp_np)

    FT, G, C = pl.pallas_call(
        functools.partial(_pre_body, n_real=n),
        out_shape=[
            jax.ShapeDtypeStruct((_HCH, np_), f32),
            jax.ShapeDtypeStruct((np_, 3 * _CUTS), f32),
            jax.ShapeDtypeStruct((np_, _CUTS), f32),
        ],
    )(featT, nuv9, P, M, Sp, W_in1, b_in1.reshape(-1, 1), W_in2,
      b_in2.reshape(-1, 1), g_in.reshape(-1, 1), be_in.reshape(-1, 1),
      Cb1.reshape(1, -1))

    ones_col = jnp.ones((np_, 1), f32)
    P4 = jnp.concatenate([2.0 * P, -SQ], axis=1)
    PT4 = jnp.concatenate([PT, jnp.ones((1, np_), f32)], axis=0)
    NI4 = jnp.concatenate([-NI, ones_col], axis=1)
    NT4 = jnp.concatenate([NT, 2.0 * jnp.ones((1, np_), f32)], axis=0)
    SQTn = -SQT
    G4 = jnp.concatenate(
        sum([[G[:, 3 * k:3 * k + 3], C[:, k:k + 1]] for k in range(_CUTS)],
            []), axis=1)

    grid = (np_ // _BI, np_ // _BJ)
    conv = pl.pallas_call(
        _conv_body,
        grid=grid,
        in_specs=[
            pl.BlockSpec((_BI, 4), lambda i, j: (i, 0)),
            pl.BlockSpec((4, _BJ), lambda i, j: (0, j)),
            pl.BlockSpec((_BI, 4), lambda i, j: (i, 0)),
            pl.BlockSpec((4, _BJ), lambda i, j: (0, j)),
            pl.BlockSpec((1, _BJ), lambda i, j: (0, j)),
            pl.BlockSpec((_BI, 4 * _CUTS), lambda i, j: (i, 0)),
            pl.BlockSpec((_HCH, _BJ), lambda i, j: (0, j)),
            pl.BlockSpec(memory_space=pltpu.SMEM),
            pl.BlockSpec(memory_space=pltpu.SMEM),
        ],
        out_specs=pl.BlockSpec((_BI, _HCH), lambda i, j: (i, 0)),
        out_shape=jax.ShapeDtypeStruct((np_, _HCH), f32),
    )(P4, PT4, NI4, NT4, SQTn, G4, FT.astype(jnp.bfloat16), Cw2,
      Cb2.reshape(-1, 1))

    outp = pl.pallas_call(
        functools.partial(_post_body, n_real=n),
        out_shape=jax.ShapeDtypeStruct((np_, och), f32),
    )(conv, W_out1.T, b_out1.reshape(1, -1), W_out2.T,
      b_out2.reshape(1, -1), g_out.reshape(1, -1), be_out.reshape(1, -1))
    return outp[:n]


# speedup vs baseline: 2.2004x; 1.0021x over previous
import functools
from math import sqrt

import jax
import jax.numpy as jnp
import numpy as np
from jax.experimental import pallas as pl
from jax.experimental.pallas import tpu as pltpu

_RADIUS = 9.0
_LEAK = 0.2
_EPS = 1e-5
_BI = 256
_BJ = 2048
_SUB = 128
_SUBI = 32
_PADJ = 2048
_CUTS = 8
_HCH = 16


def _leaky(x):
    return jnp.where(x >= 0, x, _LEAK * x)


def _pre_body(featT, nuv9, P, M, Sp, W1, b1, W2, b2, g, be, Cb1,
              FT_out, G_out, C_out, *, n_real):
    np_ = featT.shape[1]
    t1 = _leaky(jnp.dot(W1[...], featT[...],
                        preferred_element_type=jnp.float32) + b1[...])
    t2 = _leaky(jnp.dot(W2[...], t1,
                        preferred_element_type=jnp.float32) + b2[...])
    mask = (jax.lax.broadcasted_iota(jnp.int32, (1, np_), 1)
            < n_real).astype(jnp.float32)
    t2 = t2 * mask
    den = 4.0 * n_real
    groups = []
    for gi in range(4):
        sub = t2[4 * gi:4 * gi + 4, :]
        s1 = jnp.sum(sub, keepdims=True)
        s2 = jnp.sum(sub * sub, keepdims=True)
        mean = s1 / den
        var = s2 / den - mean * mean
        groups.append((sub - mean) * jax.lax.rsqrt(var + _EPS))
    norm = jnp.concatenate(groups, axis=0)
    FT_out[...] = (norm * g[...] + be[...]) * mask
    Gc = jnp.dot(nuv9[...], M[...], preferred_element_type=jnp.float32)
    G_out[...] = Gc
    Pt = jnp.concatenate([P[...]] * _CUTS, axis=1)
    C_out[...] = Cb1[...] - jnp.dot(Gc * Pt, Sp[...],
                                    preferred_element_type=jnp.float32)


def _conv_body(P4, PT4, NI4, NT4, SQTn, G4, FTb, Cw2s, Cb2s, out):
    j = pl.program_id(1)
    bi = P4.shape[0]
    bj = PT4.shape[1]
    lhs = jnp.concatenate(
        [P4[...]] + [G4[:, 4 * k:4 * k + 4] for k in range(_CUTS)], axis=0)
    R = jnp.dot(lhs, PT4[...], preferred_element_type=jnp.float32)
    tmat = jnp.dot(NI4[...], NT4[...], preferred_element_type=jnp.float32)
    zero_bf = jnp.bfloat16(0)
    cw_bf = [[jnp.bfloat16(Cw2s[d, k]) for k in range(_CUTS)]
             for d in range(_HCH)]
    cb_bf = [jnp.bfloat16(Cb2s[d, 0]) for d in range(_HCH)]
    accs = [None] * (bi // _SUBI)
    for c in range(bj // _SUB):
        sl = slice(c * _SUB, (c + 1) * _SUB)
        for ii in range(bi // _SUBI):
            rsl = slice(ii * _SUBI, (ii + 1) * _SUBI)
            neg_sumsq = R[ii * _SUBI:(ii + 1) * _SUBI, sl] + SQTn[:, sl]
            t = tmat[rsl, sl]
            w_bf = jnp.exp(neg_sumsq * t * t).astype(jnp.bfloat16)
            Ys = [jnp.maximum(
                R[(k + 1) * bi + ii * _SUBI:(k + 1) * bi + (ii + 1) * _SUBI,
                  sl].astype(jnp.bfloat16), zero_bf)
                for k in range(_CUTS)]
            cols = []
            for d in range(_HCH):
                terms = [Ys[k] * cw_bf[d][k] for k in range(_CUTS)]
                while len(terms) > 1:
                    terms = [terms[t] + terms[t + 1]
                             for t in range(0, len(terms) - 1, 2)] + \
                            (terms[-1:] if len(terms) % 2 else [])
                s = terms[0]
                z = jnp.maximum(s + cb_bf[d], zero_bf)
                prod = (z * (w_bf * FTb[d:d + 1, sl])).astype(jnp.float32)
                cols.append(jnp.sum(prod, axis=1, keepdims=True))
            part = jnp.concatenate(cols, axis=1)
            accs[ii] = part if accs[ii] is None else accs[ii] + part
    acc = jnp.concatenate(accs, axis=0)

    @pl.when(j == 0)
    def _():
        out[...] = jnp.zeros_like(out)

    out[...] += acc


def _post_body(X, W1T, b1, W2T, b2, g, be, out, *, n_real):
    np_ = X.shape[0]
    h = _leaky(jnp.dot(X[...], W1T[...],
                       preferred_element_type=jnp.float32) + b1[...])
    h = _leaky(jnp.dot(h, W2T[...],
                       preferred_element_type=jnp.float32) + b2[...])
    rmask = (jax.lax.broadcasted_iota(jnp.int32, (np_, 1), 0)
             < n_real).astype(jnp.float32)
    hm = h * rmask
    och = h.shape[1]
    gch = och // 4
    den = float(gch) * n_real
    groups = []
    for gi in range(4):
        sub = hm[:, gch * gi:gch * gi + gch]
        s1 = jnp.sum(sub, keepdims=True)
        s2 = jnp.sum(sub * sub, keepdims=True)
        mean = s1 / den
        var = s2 / den - mean * mean
        groups.append((sub - mean) * jax.lax.rsqrt(var + _EPS))
    norm = jnp.concatenate(groups, axis=1)
    out[...] = norm * g[...] + be[...]


def kernel(points, nuv, features, W_in1, b_in1, W_in2, b_in2, g_in, be_in,
           Cw1, Cb1, Cw2, Cb2, W_out1, b_out1, W_out2, b_out2, g_out, be_out):
    n = points.shape[0]
    och = W_out1.shape[0]
    np_ = ((n + _PADJ - 1) // _PADJ) * _PADJ
    pad = np_ - n
    f32 = jnp.float32

    p = (points * (1.0 / (sqrt(2.0) * _RADIUS))).astype(f32)
    P = jnp.pad(p, ((0, pad), (0, 0)))
    PT = P.T
    NI = jnp.pad(nuv[:, 0, :], ((0, pad), (0, 0)))
    NT = NI.T
    SQ = jnp.sum(P * P, axis=1, keepdims=True)
    SQT = SQ.T
    nuv9 = jnp.pad(nuv.reshape(n, 9), ((0, pad), (0, 0)))
    featT = jnp.pad(features.T, ((0, 0), (0, pad)))

    rows, colsx, kk, aa = [], [], [], []
    for k in range(_CUTS):
        for a in range(3):
            for b in range(3):
                rows.append(3 * a + b)
                colsx.append(3 * k + b)
                kk.append(k)
                aa.append(a)
    M = jnp.zeros((9, 3 * _CUTS), f32).at[
        jnp.array(rows), jnp.array(colsx)].set(Cw1[jnp.array(kk), jnp.array(aa)])
    Sp_np = np.zeros((3 * _CUTS, _CUTS), np.float32)
    for k in range(_CUTS):
        for b in range(3):
            Sp_np[3 * k + b, k] = 1.0
    Sp = jnp.asarray(Sp_np)

    FT, G, C = pl.pallas_call(
        functools.partial(_pre_body, n_real=n),
        out_shape=[
            jax.ShapeDtypeStruct((_HCH, np_), f32),
            jax.ShapeDtypeStruct((np_, 3 * _CUTS), f32),
            jax.ShapeDtypeStruct((np_, _CUTS), f32),
        ],
    )(featT, nuv9, P, M, Sp, W_in1, b_in1.reshape(-1, 1), W_in2,
      b_in2.reshape(-1, 1), g_in.reshape(-1, 1), be_in.reshape(-1, 1),
      Cb1.reshape(1, -1))

    ones_col = jnp.ones((np_, 1), f32)
    P4 = jnp.concatenate([2.0 * P, -SQ], axis=1)
    PT4 = jnp.concatenate([PT, jnp.ones((1, np_), f32)], axis=0)
    NI4 = jnp.concatenate([-NI, ones_col], axis=1)
    NT4 = jnp.concatenate([NT, 2.0 * jnp.ones((1, np_), f32)], axis=0)
    SQTn = -SQT
    G4 = jnp.concatenate(
        sum([[G[:, 3 * k:3 * k + 3], C[:, k:k + 1]] for k in range(_CUTS)],
            []), axis=1)

    grid = (np_ // _BI, np_ // _BJ)
    conv = pl.pallas_call(
        _conv_body,
        grid=grid,
        in_specs=[
            pl.BlockSpec((_BI, 4), lambda i, j: (i, 0)),
            pl.BlockSpec((4, _BJ), lambda i, j: (0, j)),
            pl.BlockSpec((_BI, 4), lambda i, j: (i, 0)),
            pl.BlockSpec((4, _BJ), lambda i, j: (0, j)),
            pl.BlockSpec((1, _BJ), lambda i, j: (0, j)),
            pl.BlockSpec((_BI, 4 * _CUTS), lambda i, j: (i, 0)),
            pl.BlockSpec((_HCH, _BJ), lambda i, j: (0, j)),
            pl.BlockSpec(memory_space=pltpu.SMEM),
            pl.BlockSpec(memory_space=pltpu.SMEM),
        ],
        out_specs=pl.BlockSpec((_BI, _HCH), lambda i, j: (i, 0)),
        out_shape=jax.ShapeDtypeStruct((np_, _HCH), f32),
    )(P4, PT4, NI4, NT4, SQTn, G4, FT.astype(jnp.bfloat16), Cw2,
      Cb2.reshape(-1, 1))

    outp = pl.pallas_call(
        functools.partial(_post_body, n_real=n),
        out_shape=jax.ShapeDtypeStruct((np_, och), f32),
    )(conv, W_out1.T, b_out1.reshape(1, -1), W_out2.T,
      b_out2.reshape(1, -1), g_out.reshape(1, -1), be_out.reshape(1, -1))
    return outp[:n]
